# baseline (speedup 1.0000x reference)
import functools
import math

import jax
import jax.numpy as jnp
from jax import lax
from jax.experimental import pallas as pl
from jax.experimental.pallas import tpu as pltpu
from jax.experimental.pallas import tpu_sc as plsc

VOCAB = 1000000
D = 64
BATCH = 1024
SEQ = 200

NC = 2
NS = 16
NW = NC * NS

NPOS = BATCH * SEQ
CHUNK = 128
CPW = NPOS // (NW * CHUNK)
POS_PER_W = CPW * CHUNK

_SCALE = math.sqrt(D)


def _pe_body(out_ref):
    pos = lax.broadcasted_iota(jnp.int32, (SEQ, D), 0).astype(jnp.float32)
    i = lax.broadcasted_iota(jnp.int32, (SEQ, D), 1)
    two_i = (2 * (i // 2)).astype(jnp.float32)
    inv_rate = jnp.exp(-(math.log(10000.0) / D) * two_i)
    angles = pos * inv_rate
    even = (i % 2) == 0
    out_ref[...] = jnp.where(even, jnp.sin(angles), jnp.cos(angles))


def _positional_encoding():
    return pl.pallas_call(
        _pe_body,
        out_shape=jax.ShapeDtypeStruct((SEQ, D), jnp.float32),
    )()


def _sc_body(x_hbm, t2_hbm, pe_hbm, out_hbm,
             idx_v, pair_v, paroff_v, g0, g1, o0, o1, pe_v,
             gsem0, gsem1, osem0, osem1):
    cid = lax.axis_index("c")
    sid = lax.axis_index("s")
    wid = sid * NC + cid
    p0 = wid * POS_PER_W

    gbufs = (g0, g1)
    gsems = (gsem0, gsem1)
    obufs = (o0, o1)
    osems = (osem0, osem1)

    pltpu.sync_copy(x_hbm.at[pl.ds(p0, POS_PER_W)], idx_v)
    pltpu.sync_copy(pe_hbm, pe_v)

    def prep(i, c2):
        sl = (pl.ds(i * 16, 16),)
        v = idx_v[sl]
        pair_v[sl] = v >> 1
        paroff_v[sl] = (v & 1) << 6
        return c2

    lax.fori_loop(0, POS_PER_W // 16, prep, 0)

    def gather_desc(t, p):
        return pltpu.make_async_copy(
            t2_hbm.at[pair_v.at[pl.ds(t * CHUNK, CHUNK)]], gbufs[p], gsems[p])

    def out_desc(t, p):
        return pltpu.make_async_copy(
            obufs[p], out_hbm.at[pl.ds(p0 + t * CHUNK, CHUNK)], osems[p])

    def compute(t, p):
        gbuf = gbufs[p]
        obuf = obufs[p]

        def group(g, c2):
            base = t * CHUNK + g * 16
            parvec = paroff_v[pl.ds(base, 16)]
            for j in range(16):
                col = parvec[j]
                r = g * 16 + j
                s = lax.rem(base + j, SEQ)
                for c in range(D // 16):
                    obuf[r, pl.ds(c * 16, 16)] = (
                        gbuf[r, pl.ds(col + c * 16, 16)] * _SCALE
                        + pe_v[s, pl.ds(c * 16, 16)])
            return c2

        lax.fori_loop(0, CHUNK // 16, group, 0)

    gather_desc(0, 0).start()

    def chunk_pair(u, c2):
        for p in (0, 1):
            t = 2 * u + p

            if p == 0:
                gather_desc(t + 1, 1).start()
            else:
                @pl.when(u < CPW // 2 - 1)
                def _():
                    gather_desc(t + 1, 0).start()

            gather_desc(t, p).wait()

            @pl.when(u > 0)
            def _():
                out_desc(t - 2, p).wait()

            compute(t, p)
            out_desc(t, p).start()
        return c2

    lax.fori_loop(0, CPW // 2, chunk_pair, 0)
    out_desc(CPW - 2, 0).wait()
    out_desc(CPW - 1, 1).wait()


@jax.jit
def _embed(x2, t2, pe):
    mesh = plsc.VectorSubcoreMesh(core_axis_name="c", subcore_axis_name="s")
    fn = functools.partial(
        pl.kernel,
        mesh=mesh,
        out_type=jax.ShapeDtypeStruct((NPOS, D), jnp.float32),
        scratch_types=[
            pltpu.VMEM((POS_PER_W,), jnp.int32),
            pltpu.VMEM((POS_PER_W,), jnp.int32),
            pltpu.VMEM((POS_PER_W,), jnp.int32),
            pltpu.VMEM((CHUNK, 2 * D), jnp.float32),
            pltpu.VMEM((CHUNK, 2 * D), jnp.float32),
            pltpu.VMEM((CHUNK, D), jnp.float32),
            pltpu.VMEM((CHUNK, D), jnp.float32),
            pltpu.VMEM((SEQ, D), jnp.float32),
            pltpu.SemaphoreType.DMA,
            pltpu.SemaphoreType.DMA,
            pltpu.SemaphoreType.DMA,
            pltpu.SemaphoreType.DMA,
        ],
        compiler_params=pltpu.CompilerParams(use_tc_tiling_on_sc=True),
    )(_sc_body)
    return fn(x2, t2, pe)


def kernel(x, table):
    pe = _positional_encoding()
    x2 = x.reshape(NPOS)
    t2 = table.reshape(VOCAB // 2, 2 * D)
    return _embed(x2, t2, pe).reshape(BATCH, SEQ, D)


# speedup vs baseline: 1.2069x; 1.2069x over previous
import functools
import math

import jax
import jax.numpy as jnp
from jax import lax
from jax.experimental import pallas as pl
from jax.experimental.pallas import tpu as pltpu
from jax.experimental.pallas import tpu_sc as plsc

VOCAB = 1000000
D = 64
BATCH = 1024
SEQ = 200

NC = 2
NS = 16
NW = NC * NS

NPOS = BATCH * SEQ
CHUNK = 128
CPW = NPOS // (NW * CHUNK)
POS_PER_W = CPW * CHUNK

_SCALE = math.sqrt(D)


def _pe_body(out_ref):
    pos = lax.broadcasted_iota(jnp.int32, (SEQ, D), 0).astype(jnp.float32)
    i = lax.broadcasted_iota(jnp.int32, (SEQ, D), 1)
    two_i = (2 * (i // 2)).astype(jnp.float32)
    inv_rate = jnp.exp(-(math.log(10000.0) / D) * two_i)
    angles = pos * inv_rate
    even = (i % 2) == 0
    out_ref[...] = jnp.where(even, jnp.sin(angles), jnp.cos(angles))


def _positional_encoding():
    return pl.pallas_call(
        _pe_body,
        out_shape=jax.ShapeDtypeStruct((SEQ, D), jnp.float32),
    )()


H = 512000
TW = 1024


def _t2_body(a_ref, b_ref, out_ref):
    out_ref[:, 0:D] = jnp.transpose(a_ref[...])
    out_ref[:, D:2 * D] = jnp.transpose(b_ref[...])


def _pack_table(tview):
    nb_b = VOCAB // TW

    return pl.pallas_call(
        _t2_body,
        grid=(H // TW,),
        in_specs=[
            pl.BlockSpec((D, TW), lambda j: (0, j)),
            pl.BlockSpec((D, TW), lambda j: (0, jnp.minimum(j + H // TW, nb_b))),
        ],
        out_specs=pl.BlockSpec((TW, 2 * D), lambda j: (j, 0)),
        out_shape=jax.ShapeDtypeStruct((H, 2 * D), jnp.float32),
    )(tview, tview)


def _sc_body(x_hbm, t2_hbm, pe_hbm, out_hbm,
             idx_v, pair_v, paroff_v, g0, g1, o0, o1, pe_v,
             gsem0, gsem1, osem0, osem1):
    cid = lax.axis_index("c")
    sid = lax.axis_index("s")
    wid = sid * NC + cid
    p0 = wid * POS_PER_W

    gbufs = (g0, g1)
    gsems = (gsem0, gsem1)
    obufs = (o0, o1)
    osems = (osem0, osem1)

    pltpu.sync_copy(x_hbm.at[pl.ds(p0, POS_PER_W)], idx_v)
    pltpu.sync_copy(pe_hbm, pe_v)

    def prep(i, c2):
        sl = (pl.ds(i * 16, 16),)
        v = idx_v[sl]
        nh = ~((v - H) >> 31)
        pair_v[sl] = v - (nh & H)
        paroff_v[sl] = nh & 64
        return c2

    lax.fori_loop(0, POS_PER_W // 16, prep, 0)

    def gather_desc(t, p):
        return pltpu.make_async_copy(
            t2_hbm.at[pair_v.at[pl.ds(t * CHUNK, CHUNK)]], gbufs[p], gsems[p])

    def out_desc(t, p):
        return pltpu.make_async_copy(
            obufs[p], out_hbm.at[pl.ds(p0 + t * CHUNK, CHUNK)], osems[p])

    def compute(t, p):
        gbuf = gbufs[p]
        obuf = obufs[p]

        def group(g, c2):
            base = t * CHUNK + g * 16
            parvec = paroff_v[pl.ds(base, 16)]
            for j in range(16):
                col = parvec[j]
                r = g * 16 + j
                s = lax.rem(base + j, SEQ)
                for c in range(D // 16):
                    obuf[r, pl.ds(c * 16, 16)] = (
                        gbuf[r, pl.ds(col + c * 16, 16)] * _SCALE
                        + pe_v[s, pl.ds(c * 16, 16)])
            return c2

        lax.fori_loop(0, CHUNK // 16, group, 0)

    gather_desc(0, 0).start()

    def chunk_pair(u, c2):
        for p in (0, 1):
            t = 2 * u + p

            if p == 0:
                gather_desc(t + 1, 1).start()
            else:
                @pl.when(u < CPW // 2 - 1)
                def _():
                    gather_desc(t + 1, 0).start()

            gather_desc(t, p).wait()

            @pl.when(u > 0)
            def _():
                out_desc(t - 2, p).wait()

            compute(t, p)
            out_desc(t, p).start()
        return c2

    lax.fori_loop(0, CPW // 2, chunk_pair, 0)
    out_desc(CPW - 2, 0).wait()
    out_desc(CPW - 1, 1).wait()


@jax.jit
def _embed(x2, t2, pe):
    mesh = plsc.VectorSubcoreMesh(core_axis_name="c", subcore_axis_name="s")
    fn = functools.partial(
        pl.kernel,
        mesh=mesh,
        out_type=jax.ShapeDtypeStruct((NPOS, D), jnp.float32),
        scratch_types=[
            pltpu.VMEM((POS_PER_W,), jnp.int32),
            pltpu.VMEM((POS_PER_W,), jnp.int32),
            pltpu.VMEM((POS_PER_W,), jnp.int32),
            pltpu.VMEM((CHUNK, 2 * D), jnp.float32),
            pltpu.VMEM((CHUNK, 2 * D), jnp.float32),
            pltpu.VMEM((CHUNK, D), jnp.float32),
            pltpu.VMEM((CHUNK, D), jnp.float32),
            pltpu.VMEM((SEQ, D), jnp.float32),
            pltpu.SemaphoreType.DMA,
            pltpu.SemaphoreType.DMA,
            pltpu.SemaphoreType.DMA,
            pltpu.SemaphoreType.DMA,
        ],
        compiler_params=pltpu.CompilerParams(use_tc_tiling_on_sc=True),
    )(_sc_body)
    return fn(x2, t2, pe)


def kernel(x, table):
    pe = _positional_encoding()
    x2 = x.reshape(NPOS)
    t2 = _pack_table(jnp.transpose(table))
    return _embed(x2, t2, pe).reshape(BATCH, SEQ, D)


# speedup vs baseline: 1.4740x; 1.2213x over previous
import functools
import math

import jax
import jax.numpy as jnp
from jax import lax
from jax.experimental import pallas as pl
from jax.experimental.pallas import tpu as pltpu
from jax.experimental.pallas import tpu_sc as plsc

VOCAB = 1000000
D = 64
BATCH = 1024
SEQ = 200

NC = 2
NS = 16
NW = NC * NS

NPOS = BATCH * SEQ
CHUNK = 128
CPW = NPOS // (NW * CHUNK)
POS_PER_W = CPW * CHUNK

_SCALE = math.sqrt(D)


def _pe_body(out_ref):
    pos = lax.broadcasted_iota(jnp.int32, (SEQ, D), 0).astype(jnp.float32)
    i = lax.broadcasted_iota(jnp.int32, (SEQ, D), 1)
    two_i = (2 * (i // 2)).astype(jnp.float32)
    inv_rate = jnp.exp(-(math.log(10000.0) / D) * two_i)
    angles = pos * inv_rate
    even = (i % 2) == 0
    out_ref[...] = jnp.where(even, jnp.sin(angles), jnp.cos(angles))


def _positional_encoding():
    return pl.pallas_call(
        _pe_body,
        out_shape=jax.ShapeDtypeStruct((SEQ, D), jnp.float32),
    )()


H = 512000
TW = 2048


def _t2_body(a_ref, b_ref, out_ref):
    r = lax.broadcasted_iota(jnp.int32, (D, D), 0)
    c = lax.broadcasted_iota(jnp.int32, (D, D), 1)
    ident = jnp.where(r == c, 1.0, 0.0).astype(jnp.float32)
    dn = (((0,), (0,)), ((), ()))
    out_ref[:, 0:D] = lax.dot_general(
        a_ref[...], ident, dn, preferred_element_type=jnp.float32)
    out_ref[:, D:2 * D] = lax.dot_general(
        b_ref[...], ident, dn, preferred_element_type=jnp.float32)


def _pack_table(tview):
    nb_b = VOCAB // TW

    return pl.pallas_call(
        _t2_body,
        grid=(H // TW,),
        in_specs=[
            pl.BlockSpec((D, TW), lambda j: (0, j)),
            pl.BlockSpec((D, TW), lambda j: (0, jnp.minimum(j + H // TW, nb_b))),
        ],
        out_specs=pl.BlockSpec((TW, 2 * D), lambda j: (j, 0)),
        out_shape=jax.ShapeDtypeStruct((H, 2 * D), jnp.float32),
    )(tview, tview)


def _sc_body(x_hbm, t2_hbm, pe_hbm, out_hbm,
             idx_v, pair_v, paroff_v, g0, g1, o0, o1, pe_v,
             gsem0, gsem1, osem0, osem1):
    cid = lax.axis_index("c")
    sid = lax.axis_index("s")
    wid = sid * NC + cid
    p0 = wid * POS_PER_W

    gbufs = (g0, g1)
    gsems = (gsem0, gsem1)
    obufs = (o0, o1)
    osems = (osem0, osem1)

    pltpu.sync_copy(x_hbm.at[pl.ds(p0, POS_PER_W)], idx_v)
    pltpu.sync_copy(pe_hbm, pe_v)

    def prep(i, c2):
        sl = (pl.ds(i * 16, 16),)
        v = idx_v[sl]
        nh = ~((v - H) >> 31)
        pair_v[sl] = v - (nh & H)
        paroff_v[sl] = nh & 64
        return c2

    lax.fori_loop(0, POS_PER_W // 16, prep, 0)

    def gather_desc(t, p):
        return pltpu.make_async_copy(
            t2_hbm.at[pair_v.at[pl.ds(t * CHUNK, CHUNK)]], gbufs[p], gsems[p])

    def out_desc(t, p):
        return pltpu.make_async_copy(
            obufs[p], out_hbm.at[pl.ds(p0 + t * CHUNK, CHUNK)], osems[p])

    def compute(t, p):
        gbuf = gbufs[p]
        obuf = obufs[p]

        def group(g, c2):
            base = t * CHUNK + g * 16
            parvec = paroff_v[pl.ds(base, 16)]
            for j in range(16):
                col = parvec[j]
                r = g * 16 + j
                s = lax.rem(base + j, SEQ)
                for c in range(D // 16):
                    obuf[r, pl.ds(c * 16, 16)] = (
                        gbuf[r, pl.ds(col + c * 16, 16)] * _SCALE
                        + pe_v[s, pl.ds(c * 16, 16)])
            return c2

        lax.fori_loop(0, CHUNK // 16, group, 0)

    gather_desc(0, 0).start()

    def chunk_pair(u, c2):
        for p in (0, 1):
            t = 2 * u + p

            if p == 0:
                gather_desc(t + 1, 1).start()
            else:
                @pl.when(u < CPW // 2 - 1)
                def _():
                    gather_desc(t + 1, 0).start()

            gather_desc(t, p).wait()

            @pl.when(u > 0)
            def _():
                out_desc(t - 2, p).wait()

            compute(t, p)
            out_desc(t, p).start()
        return c2

    lax.fori_loop(0, CPW // 2, chunk_pair, 0)
    out_desc(CPW - 2, 0).wait()
    out_desc(CPW - 1, 1).wait()


@jax.jit
def _embed(x2, t2, pe):
    mesh = plsc.VectorSubcoreMesh(core_axis_name="c", subcore_axis_name="s")
    fn = functools.partial(
        pl.kernel,
        mesh=mesh,
        out_type=jax.ShapeDtypeStruct((NPOS, D), jnp.float32),
        scratch_types=[
            pltpu.VMEM((POS_PER_W,), jnp.int32),
            pltpu.VMEM((POS_PER_W,), jnp.int32),
            pltpu.VMEM((POS_PER_W,), jnp.int32),
            pltpu.VMEM((CHUNK, 2 * D), jnp.float32),
            pltpu.VMEM((CHUNK, 2 * D), jnp.float32),
            pltpu.VMEM((CHUNK, D), jnp.float32),
            pltpu.VMEM((CHUNK, D), jnp.float32),
            pltpu.VMEM((SEQ, D), jnp.float32),
            pltpu.SemaphoreType.DMA,
            pltpu.SemaphoreType.DMA,
            pltpu.SemaphoreType.DMA,
            pltpu.SemaphoreType.DMA,
        ],
        compiler_params=pltpu.CompilerParams(use_tc_tiling_on_sc=True),
    )(_sc_body)
    return fn(x2, t2, pe)


def kernel(x, table):
    pe = _positional_encoding()
    x2 = x.reshape(NPOS)
    t2 = _pack_table(jnp.transpose(table))
    return _embed(x2, t2, pe).reshape(BATCH, SEQ, D)


# speedup vs baseline: 1.6806x; 1.1402x over previous
import functools
import math

import jax
import jax.numpy as jnp
from jax import lax
from jax.experimental import pallas as pl
from jax.experimental.pallas import tpu as pltpu
from jax.experimental.pallas import tpu_sc as plsc

VOCAB = 1000000
D = 64
BATCH = 1024
SEQ = 200

NC = 2
NS = 16
NW = NC * NS

NPOS = BATCH * SEQ
CHUNK = 128
CPW = NPOS // (NW * CHUNK)
POS_PER_W = CPW * CHUNK

_SCALE = math.sqrt(D)


def _pe_body(out_ref):
    pos = lax.broadcasted_iota(jnp.int32, (SEQ, D), 0).astype(jnp.float32)
    i = lax.broadcasted_iota(jnp.int32, (SEQ, D), 1)
    two_i = (2 * (i // 2)).astype(jnp.float32)
    inv_rate = jnp.exp(-(math.log(10000.0) / D) * two_i)
    angles = pos * inv_rate
    even = (i % 2) == 0
    out_ref[...] = jnp.where(even, jnp.sin(angles), jnp.cos(angles))


def _positional_encoding():
    return pl.pallas_call(
        _pe_body,
        out_shape=jax.ShapeDtypeStruct((SEQ, D), jnp.float32),
    )()


H = 512000
TW = 4096


def _t2_body(a_ref, b_ref, out_ref):
    r = lax.broadcasted_iota(jnp.int32, (D, D), 0)
    c = lax.broadcasted_iota(jnp.int32, (D, D), 1)
    ident = jnp.where(r == c, 1.0, 0.0).astype(jnp.float32)
    dn = (((0,), (0,)), ((), ()))
    out_ref[:, 0:D] = lax.dot_general(
        a_ref[...], ident, dn, preferred_element_type=jnp.float32)
    out_ref[:, D:2 * D] = lax.dot_general(
        b_ref[...], ident, dn, preferred_element_type=jnp.float32)


def _pack_table(tview):
    nb_b = VOCAB // TW

    return pl.pallas_call(
        _t2_body,
        grid=(H // TW,),
        in_specs=[
            pl.BlockSpec((D, TW), lambda j: (0, j)),
            pl.BlockSpec((D, TW), lambda j: (0, jnp.minimum(j + H // TW, nb_b))),
        ],
        out_specs=pl.BlockSpec((TW, 2 * D), lambda j: (j, 0)),
        out_shape=jax.ShapeDtypeStruct((H, 2 * D), jnp.float32),
    )(tview, tview)


def _sc_body(x_hbm, t2_hbm, pe_hbm, out_hbm,
             idx_v, pair_v, paroff_v, g0, g1, o0, o1, pe_v,
             gsem0, gsem1, osem0, osem1):
    cid = lax.axis_index("c")
    sid = lax.axis_index("s")
    wid = sid * NC + cid
    p0 = wid * POS_PER_W

    gbufs = (g0, g1)
    gsems = (gsem0, gsem1)
    obufs = (o0, o1)
    osems = (osem0, osem1)

    pltpu.sync_copy(x_hbm.at[pl.ds(p0, POS_PER_W)], idx_v)
    pltpu.sync_copy(pe_hbm, pe_v)

    def prep(i, c2):
        sl = (pl.ds(i * 16, 16),)
        v = idx_v[sl]
        nh = ~((v - H) >> 31)
        pair_v[sl] = v - (nh & H)
        paroff_v[sl] = nh & 64
        return c2

    lax.fori_loop(0, POS_PER_W // 16, prep, 0)

    def gather_desc(t, p):
        return pltpu.make_async_copy(
            t2_hbm.at[pair_v.at[pl.ds(t * CHUNK, CHUNK)]], gbufs[p], gsems[p])

    def out_desc(t, p):
        return pltpu.make_async_copy(
            obufs[p], out_hbm.at[pl.ds(p0 + t * CHUNK, CHUNK)], osems[p])

    def compute(t, p):
        gbuf = gbufs[p]
        obuf = obufs[p]

        def group(g, c2):
            base = t * CHUNK + g * 16
            parvec = paroff_v[pl.ds(base, 16)]
            for j in range(16):
                col = parvec[j]
                r = g * 16 + j
                s = lax.rem(base + j, SEQ)
                for c in range(D // 16):
                    obuf[r, pl.ds(c * 16, 16)] = (
                        gbuf[r, pl.ds(col + c * 16, 16)] * _SCALE
                        + pe_v[s, pl.ds(c * 16, 16)])
            return c2

        lax.fori_loop(0, CHUNK // 16, group, 0)

    gather_desc(0, 0).start()

    def chunk_pair(u, c2):
        for p in (0, 1):
            t = 2 * u + p

            if p == 0:
                gather_desc(t + 1, 1).start()
            else:
                @pl.when(u < CPW // 2 - 1)
                def _():
                    gather_desc(t + 1, 0).start()

            gather_desc(t, p).wait()

            @pl.when(u > 0)
            def _():
                out_desc(t - 2, p).wait()

            compute(t, p)
            out_desc(t, p).start()
        return c2

    lax.fori_loop(0, CPW // 2, chunk_pair, 0)
    out_desc(CPW - 2, 0).wait()
    out_desc(CPW - 1, 1).wait()


@jax.jit
def _embed(x2, t2, pe):
    mesh = plsc.VectorSubcoreMesh(core_axis_name="c", subcore_axis_name="s")
    fn = functools.partial(
        pl.kernel,
        mesh=mesh,
        out_type=jax.ShapeDtypeStruct((NPOS, D), jnp.float32),
        scratch_types=[
            pltpu.VMEM((POS_PER_W,), jnp.int32),
            pltpu.VMEM((POS_PER_W,), jnp.int32),
            pltpu.VMEM((POS_PER_W,), jnp.int32),
            pltpu.VMEM((CHUNK, 2 * D), jnp.float32),
            pltpu.VMEM((CHUNK, 2 * D), jnp.float32),
            pltpu.VMEM((CHUNK, D), jnp.float32),
            pltpu.VMEM((CHUNK, D), jnp.float32),
            pltpu.VMEM((SEQ, D), jnp.float32),
            pltpu.SemaphoreType.DMA,
            pltpu.SemaphoreType.DMA,
            pltpu.SemaphoreType.DMA,
            pltpu.SemaphoreType.DMA,
        ],
        compiler_params=pltpu.CompilerParams(use_tc_tiling_on_sc=True),
    )(_sc_body)
    return fn(x2, t2, pe)


def kernel(x, table):
    pe = _positional_encoding()
    x2 = x.reshape(NPOS)
    t2 = _pack_table(jnp.transpose(table))
    return _embed(x2, t2, pe).reshape(BATCH, SEQ, D)


# speedup vs baseline: 1.8544x; 1.1034x over previous
import functools
import math

import jax
import jax.numpy as jnp
from jax import lax
from jax.experimental import pallas as pl
from jax.experimental.pallas import tpu as pltpu
from jax.experimental.pallas import tpu_sc as plsc

VOCAB = 1000000
D = 64
BATCH = 1024
SEQ = 200

NC = 2
NS = 16
NW = NC * NS

NPOS = BATCH * SEQ
CHUNK = 128
CPW = NPOS // (NW * CHUNK)
POS_PER_W = CPW * CHUNK

_SCALE = math.sqrt(D)


def _pe_body(out_ref):
    pos = lax.broadcasted_iota(jnp.int32, (SEQ, D), 0).astype(jnp.float32)
    i = lax.broadcasted_iota(jnp.int32, (SEQ, D), 1)
    two_i = (2 * (i // 2)).astype(jnp.float32)
    inv_rate = jnp.exp(-(math.log(10000.0) / D) * two_i)
    angles = pos * inv_rate
    even = (i % 2) == 0
    out_ref[...] = jnp.where(even, jnp.sin(angles), jnp.cos(angles))


def _positional_encoding():
    return pl.pallas_call(
        _pe_body,
        out_shape=jax.ShapeDtypeStruct((SEQ, D), jnp.float32),
    )()


H = 512000
TW = 12800


def _t2_body(a_ref, b_ref, out_ref):
    r = lax.broadcasted_iota(jnp.int32, (D, D), 0)
    c = lax.broadcasted_iota(jnp.int32, (D, D), 1)
    ident = jnp.where(r == c, 1.0, 0.0).astype(jnp.float32)
    dn = (((0,), (0,)), ((), ()))
    out_ref[:, 0:D] = lax.dot_general(
        a_ref[...], ident, dn, preferred_element_type=jnp.float32)
    out_ref[:, D:2 * D] = lax.dot_general(
        b_ref[...], ident, dn, preferred_element_type=jnp.float32)


def _pack_table(tview):
    nb_b = VOCAB // TW

    return pl.pallas_call(
        _t2_body,
        grid=(H // TW,),
        in_specs=[
            pl.BlockSpec((D, TW), lambda j: (0, j)),
            pl.BlockSpec((D, TW), lambda j: (0, jnp.minimum(j + H // TW, nb_b))),
        ],
        out_specs=pl.BlockSpec((TW, 2 * D), lambda j: (j, 0)),
        out_shape=jax.ShapeDtypeStruct((H, 2 * D), jnp.float32),
    )(tview, tview)


def _sc_body(x_hbm, t2_hbm, pe_hbm, out_hbm,
             idx_v, pair_v, paroff_v, g0, g1, o0, o1, pe_v,
             gsem0, gsem1, osem0, osem1):
    cid = lax.axis_index("c")
    sid = lax.axis_index("s")
    wid = sid * NC + cid
    p0 = wid * POS_PER_W

    gbufs = (g0, g1)
    gsems = (gsem0, gsem1)
    obufs = (o0, o1)
    osems = (osem0, osem1)

    pltpu.sync_copy(x_hbm.at[pl.ds(p0, POS_PER_W)], idx_v)
    pltpu.sync_copy(pe_hbm, pe_v)

    def prep(i, c2):
        sl = (pl.ds(i * 16, 16),)
        v = idx_v[sl]
        nh = ~((v - H) >> 31)
        pair_v[sl] = v - (nh & H)
        paroff_v[sl] = nh & 64
        return c2

    lax.fori_loop(0, POS_PER_W // 16, prep, 0)

    def gather_desc(t, p):
        return pltpu.make_async_copy(
            t2_hbm.at[pair_v.at[pl.ds(t * CHUNK, CHUNK)]], gbufs[p], gsems[p])

    def out_desc(t, p):
        return pltpu.make_async_copy(
            obufs[p], out_hbm.at[pl.ds(p0 + t * CHUNK, CHUNK)], osems[p])

    def compute(t, p):
        gbuf = gbufs[p]
        obuf = obufs[p]

        def group(g, c2):
            base = t * CHUNK + g * 16
            parvec = paroff_v[pl.ds(base, 16)]
            for j in range(16):
                col = parvec[j]
                r = g * 16 + j
                s = lax.rem(base + j, SEQ)
                for c in range(D // 16):
                    obuf[r, pl.ds(c * 16, 16)] = (
                        gbuf[r, pl.ds(col + c * 16, 16)] * _SCALE
                        + pe_v[s, pl.ds(c * 16, 16)])
            return c2

        lax.fori_loop(0, CHUNK // 16, group, 0)

    gather_desc(0, 0).start()

    def chunk_pair(u, c2):
        for p in (0, 1):
            t = 2 * u + p

            if p == 0:
                gather_desc(t + 1, 1).start()
            else:
                @pl.when(u < CPW // 2 - 1)
                def _():
                    gather_desc(t + 1, 0).start()

            gather_desc(t, p).wait()

            @pl.when(u > 0)
            def _():
                out_desc(t - 2, p).wait()

            compute(t, p)
            out_desc(t, p).start()
        return c2

    lax.fori_loop(0, CPW // 2, chunk_pair, 0)
    out_desc(CPW - 2, 0).wait()
    out_desc(CPW - 1, 1).wait()


@jax.jit
def _embed(x2, t2, pe):
    mesh = plsc.VectorSubcoreMesh(core_axis_name="c", subcore_axis_name="s")
    fn = functools.partial(
        pl.kernel,
        mesh=mesh,
        out_type=jax.ShapeDtypeStruct((NPOS, D), jnp.float32),
        scratch_types=[
            pltpu.VMEM((POS_PER_W,), jnp.int32),
            pltpu.VMEM((POS_PER_W,), jnp.int32),
            pltpu.VMEM((POS_PER_W,), jnp.int32),
            pltpu.VMEM((CHUNK, 2 * D), jnp.float32),
            pltpu.VMEM((CHUNK, 2 * D), jnp.float32),
            pltpu.VMEM((CHUNK, D), jnp.float32),
            pltpu.VMEM((CHUNK, D), jnp.float32),
            pltpu.VMEM((SEQ, D), jnp.float32),
            pltpu.SemaphoreType.DMA,
            pltpu.SemaphoreType.DMA,
            pltpu.SemaphoreType.DMA,
            pltpu.SemaphoreType.DMA,
        ],
        compiler_params=pltpu.CompilerParams(use_tc_tiling_on_sc=True),
    )(_sc_body)
    return fn(x2, t2, pe)


def kernel(x, table):
    pe = _positional_encoding()
    x2 = x.reshape(NPOS)
    t2 = _pack_table(jnp.transpose(table))
    return _embed(x2, t2, pe).reshape(BATCH, SEQ, D)
